# baseline (speedup 1.0000x reference)
import functools

import jax
import jax.numpy as jnp
from jax import lax
from jax.experimental import pallas as pl
from jax.experimental.pallas import tpu as pltpu
from jax.experimental.pallas import tpu_sc as plsc

B = 1024
T = 200
D = 768


def _build():
    info = plsc.get_sparse_core_info()
    NC, NS, L = info.num_cores, info.num_subcores, info.num_lanes
    NW = NC * NS
    nrows = 2 * B // NW

    mesh = plsc.VectorSubcoreMesh(core_axis_name="c", subcore_axis_name="s")

    nchunk = nrows // L

    @functools.partial(
        pl.kernel,
        mesh=mesh,
        out_type=jax.ShapeDtypeStruct((2 * B, D), jnp.float32),
        scratch_types=[
            pltpu.VMEM((nrows,), jnp.int32),
            pltpu.VMEM((nrows, D), jnp.float32),
            [pltpu.SemaphoreType.DMA] * nchunk,
            [pltpu.SemaphoreType.DMA] * nchunk,
        ],
    )
    def k(table_hbm, idx_hbm, out_hbm, flat_v, rows_v, g_sems, w_sems):
        wid = lax.axis_index("s") * NC + lax.axis_index("c")
        base = wid * nrows
        pltpu.sync_copy(idx_hbm.at[pl.ds(base, nrows)], flat_v)
        for i in range(nchunk):
            j = base + i * L + lax.iota(jnp.int32, L)
            b = lax.shift_right_logical(j, 1)
            flat_v[pl.ds(i * L, L)] = b * T + flat_v[pl.ds(i * L, L)]
        def gather(i):
            return pltpu.async_copy(
                table_hbm.at[flat_v.at[pl.ds(i * L, L)]],
                rows_v.at[pl.ds(i * L, L)], g_sems[i])

        def write(i):
            return pltpu.async_copy(
                rows_v.at[pl.ds(i * L, L)],
                out_hbm.at[pl.ds(base + i * L, L)], w_sems[i])

        gathers = {0: gather(0)}
        writes = []
        for i in range(nchunk):
            gathers[i].wait()
            if i + 1 < nchunk:
                gathers[i + 1] = gather(i + 1)
            writes.append(write(i))
        for w in writes:
            w.wait()

    return k


_gather_kernel = _build()


def kernel(token_embeddings, indexes):
    table = token_embeddings.reshape(B * T, D)
    idx = jnp.swapaxes(indexes.astype(jnp.int32), 0, 1).reshape(2 * B)
    out = _gather_kernel(table, idx)
    return out.reshape(B, 2 * D)


# speedup vs baseline: 1.0556x; 1.0556x over previous
import functools

import jax
import jax.numpy as jnp
from jax import lax
from jax.experimental import pallas as pl
from jax.experimental.pallas import tpu as pltpu
from jax.experimental.pallas import tpu_sc as plsc

B = 1024
T = 200
D = 768


def _build():
    info = plsc.get_sparse_core_info()
    NC, NS, L = info.num_cores, info.num_subcores, info.num_lanes
    NW = NC * NS
    nrows = 2 * B // NW

    mesh = plsc.VectorSubcoreMesh(core_axis_name="c", subcore_axis_name="s")

    @functools.partial(
        pl.kernel,
        mesh=mesh,
        out_type=jax.ShapeDtypeStruct((2 * B, D), jnp.float32),
        scratch_types=[
            pltpu.VMEM((nrows,), jnp.int32),
            pltpu.VMEM((nrows, D), jnp.float32),
            pltpu.SemaphoreType.DMA,
        ],
    )
    def k(table_hbm, idx_hbm, out_hbm, flat_v, rows_v, sem):
        wid = lax.axis_index("s") * NC + lax.axis_index("c")
        base = wid * nrows
        pltpu.sync_copy(idx_hbm.at[pl.ds(base, nrows)], flat_v)
        for i in range(nrows // L):
            j = base + i * L + lax.iota(jnp.int32, L)
            b = lax.shift_right_logical(j, 1)
            flat_v[pl.ds(i * L, L)] = b * T + flat_v[pl.ds(i * L, L)]
        pltpu.async_copy(table_hbm.at[flat_v], rows_v, sem).wait()
        pltpu.sync_copy(rows_v, out_hbm.at[pl.ds(base, nrows)])

    return k


_gather_kernel = _build()


def kernel(token_embeddings, indexes):
    table = token_embeddings.reshape(B * T, D)
    idx = jnp.swapaxes(indexes.astype(jnp.int32), 0, 1).reshape(2 * B)
    out = _gather_kernel(table, idx)
    return out.reshape(B, 2 * D)
